# Initial kernel scaffold; baseline (speedup 1.0000x reference)
import jax
import jax.numpy as jnp
from jax.experimental import pallas as pl


def kernel(x, W_switch, b_switch, W1, b1, W2, b2, gamma, beta):
    raise NotImplementedError("write your pallas kernel here")


# speedup vs baseline: 1.9094x; 1.9094x over previous
import jax
import jax.numpy as jnp
from jax.experimental import pallas as pl
from jax.experimental.pallas import tpu as pltpu

_B, _S, _D, _F, _E, _K = 32, 60, 1024, 4096, 8, 2
_FT = 512
_NF = _F // _FT
_P = _B * _K


def _router_body(xf_ref, ws_ref, bs_ref, top_ref):
    logits = jnp.dot(xf_ref[...], ws_ref[...], preferred_element_type=jnp.float32)
    logits = logits + bs_ref[...]
    col = jax.lax.broadcasted_iota(jnp.int32, (_B, _E), 1)
    a1 = jnp.argmax(logits, axis=1).astype(jnp.int32)
    masked = jnp.where(col == a1[:, None], -jnp.inf, logits)
    a2 = jnp.argmax(masked, axis=1).astype(jnp.int32)
    top_ref[...] = jnp.stack([a1, a2], axis=1)


def _moe_body(pe_ref, ps_ref, x_ref, w1_ref, b1_ref, w2_ref, b2_ref, g_ref,
              bt_ref, out_ref, acc_ref):
    f = pl.program_id(0)
    p = pl.program_id(1)
    e = pe_ref[p]
    b = ps_ref[p]

    @pl.when((f == 0) & (p == 0))
    def _init():
        acc_ref[...] = jnp.zeros_like(acc_ref)

    xb = x_ref[b]
    h = jnp.dot(xb, w1_ref[0], preferred_element_type=jnp.float32)
    h = h + b1_ref[e, pl.ds(f * _FT, _FT)][None, :]
    h = 0.5 * h * (1.0 + jax.lax.erf(h * 0.7071067811865476))
    contrib = jnp.dot(h, w2_ref[0], preferred_element_type=jnp.float32)
    acc_ref[b] = acc_ref[b] + contrib

    @pl.when(f == 0)
    def _bias2():
        acc_ref[b] = acc_ref[b] + b2_ref[e][None, :]

    @pl.when((f == _NF - 1) & (p == _P - 1))
    def _finish():
        z = x_ref[...] + acc_ref[...]
        mean = jnp.mean(z, axis=-1, keepdims=True)
        zc = z - mean
        var = jnp.mean(zc * zc, axis=-1, keepdims=True)
        out_ref[...] = zc * jax.lax.rsqrt(var + 1e-5) * g_ref[0] + bt_ref[0]


def kernel(x, W_switch, b_switch, W1, b1, W2, b2, gamma, beta):
    x_flat = x.reshape(_B, _S * _D)
    top_idx = pl.pallas_call(
        _router_body,
        out_shape=jax.ShapeDtypeStruct((_B, _K), jnp.int32),
    )(x_flat, W_switch, b_switch.reshape(1, _E))

    flat_e = top_idx.reshape(-1)
    order = jnp.argsort(flat_e)
    pair_expert = flat_e[order].astype(jnp.int32)
    pair_sample = (order // _K).astype(jnp.int32)

    grid_spec = pltpu.PrefetchScalarGridSpec(
        num_scalar_prefetch=2,
        grid=(_NF, _P),
        in_specs=[
            pl.BlockSpec((_B, _S, _D), lambda f, p, pe, ps: (0, 0, 0)),
            pl.BlockSpec((1, _D, _FT), lambda f, p, pe, ps: (pe[p], 0, f)),
            pl.BlockSpec((_E, _F), lambda f, p, pe, ps: (0, 0)),
            pl.BlockSpec((1, _FT, _D), lambda f, p, pe, ps: (pe[p], f, 0)),
            pl.BlockSpec((_E, _D), lambda f, p, pe, ps: (0, 0)),
            pl.BlockSpec((1, _D), lambda f, p, pe, ps: (0, 0)),
            pl.BlockSpec((1, _D), lambda f, p, pe, ps: (0, 0)),
        ],
        out_specs=pl.BlockSpec((_B, _S, _D), lambda f, p, pe, ps: (0, 0, 0)),
        scratch_shapes=[pltpu.VMEM((_B, _S, _D), jnp.float32)],
    )
    out = pl.pallas_call(
        _moe_body,
        grid_spec=grid_spec,
        out_shape=jax.ShapeDtypeStruct((_B, _S, _D), jnp.float32),
        compiler_params=pltpu.CompilerParams(
            dimension_semantics=("arbitrary", "arbitrary")),
    )(pair_expert, pair_sample, x, W1, b1, W2, b2,
      gamma.reshape(1, _D), beta.reshape(1, _D))
    return out


# speedup vs baseline: 2.3983x; 1.2561x over previous
import jax
import jax.numpy as jnp
from jax.experimental import pallas as pl
from jax.experimental.pallas import tpu as pltpu

_B, _S, _D, _F, _E, _K = 32, 60, 1024, 4096, 8, 2
_SP = 64
_FT = 512
_NF = _F // _FT
_P = _B * _K
_PP = _P + _E
_NC = _PP // 2


def _router_body(xf_ref, ws_ref, bs_ref, top_ref):
    logits = jnp.dot(xf_ref[...], ws_ref[...], preferred_element_type=jnp.float32)
    logits = logits + bs_ref[...]
    col = jax.lax.broadcasted_iota(jnp.int32, (_B, _E), 1)
    a1 = jnp.argmax(logits, axis=1).astype(jnp.int32)
    masked = jnp.where(col == a1[:, None], -jnp.inf, logits)
    a2 = jnp.argmax(masked, axis=1).astype(jnp.int32)
    top_ref[...] = jnp.stack([a1, a2], axis=1)


def _moe_body(pe_ref, ps_ref, pv_ref, x_ref, w1_ref, b1_ref, w2_ref, b2_ref,
              g_ref, bt_ref, out_ref, acc_ref):
    f = pl.program_id(0)
    c = pl.program_id(1)
    p0 = 2 * c
    e = pe_ref[p0]
    b0 = ps_ref[p0]
    b1v = ps_ref[p0 + 1]
    v0 = pv_ref[p0]
    v1 = pv_ref[p0 + 1]

    @pl.when((f == 0) & (c == 0))
    def _init():
        acc_ref[...] = jnp.zeros_like(acc_ref)

    @pl.when(v0 > 0)
    def _compute():
        xb = jnp.concatenate([x_ref[b0], x_ref[b1v]], axis=0)
        h = jnp.dot(xb, w1_ref[0], preferred_element_type=jnp.float32)
        h = h + b1_ref[e, pl.ds(f * _FT, _FT)][None, :]
        h = 0.5 * h * (1.0 + jax.lax.erf(h * 0.7071067811865476))
        contrib = jnp.dot(h, w2_ref[0], preferred_element_type=jnp.float32)
        acc_ref[b0] = acc_ref[b0] + contrib[:_SP]

        @pl.when(v1 > 0)
        def _second():
            acc_ref[b1v] = acc_ref[b1v] + contrib[_SP:]

        @pl.when(f == 0)
        def _bias2():
            acc_ref[b0] = acc_ref[b0] + b2_ref[e][None, :]

            @pl.when(v1 > 0)
            def _bias2b():
                acc_ref[b1v] = acc_ref[b1v] + b2_ref[e][None, :]

    @pl.when((f == _NF - 1) & (c == _NC - 1))
    def _finish():
        z = x_ref[...] + acc_ref[...]
        mean = jnp.mean(z, axis=-1, keepdims=True)
        zc = z - mean
        var = jnp.mean(zc * zc, axis=-1, keepdims=True)
        res = zc * jax.lax.rsqrt(var + 1e-5) * g_ref[0] + bt_ref[0]
        out_ref[...] = res[:, :_S, :]


def kernel(x, W_switch, b_switch, W1, b1, W2, b2, gamma, beta):
    x_flat = x.reshape(_B, _S * _D)
    top_idx = pl.pallas_call(
        _router_body,
        out_shape=jax.ShapeDtypeStruct((_B, _K), jnp.int32),
    )(x_flat, W_switch, b_switch.reshape(1, _E))

    flat_e = top_idx.reshape(-1)
    order = jnp.argsort(flat_e).astype(jnp.int32)
    pe_s = flat_e[order]
    ps_s = order // _K
    counts = jnp.zeros((_E,), jnp.int32).at[flat_e].add(1)
    pad_counts = counts + (counts % 2)
    off = jnp.cumsum(counts) - counts
    off_pad = jnp.cumsum(pad_counts) - pad_counts
    pos = off_pad[pe_s] + (jnp.arange(_P, dtype=jnp.int32) - off[pe_s])
    pe_pad = jnp.zeros((_PP,), jnp.int32).at[pos].set(pe_s + 1)
    pe_pad = jnp.maximum(jax.lax.cummax(pe_pad) - 1, 0)
    ps_pad = jnp.zeros((_PP,), jnp.int32).at[pos].set(ps_s)
    pv_pad = jnp.zeros((_PP,), jnp.int32).at[pos].set(1)

    x_p = jnp.pad(x, ((0, 0), (0, _SP - _S), (0, 0)))

    grid_spec = pltpu.PrefetchScalarGridSpec(
        num_scalar_prefetch=3,
        grid=(_NF, _NC),
        in_specs=[
            pl.BlockSpec((_B, _SP, _D), lambda f, c, pe, ps, pv: (0, 0, 0)),
            pl.BlockSpec((1, _D, _FT), lambda f, c, pe, ps, pv: (pe[2 * c], 0, f)),
            pl.BlockSpec((_E, _F), lambda f, c, pe, ps, pv: (0, 0)),
            pl.BlockSpec((1, _FT, _D), lambda f, c, pe, ps, pv: (pe[2 * c], f, 0)),
            pl.BlockSpec((_E, _D), lambda f, c, pe, ps, pv: (0, 0)),
            pl.BlockSpec((1, _D), lambda f, c, pe, ps, pv: (0, 0)),
            pl.BlockSpec((1, _D), lambda f, c, pe, ps, pv: (0, 0)),
        ],
        out_specs=pl.BlockSpec((_B, _S, _D), lambda f, c, pe, ps, pv: (0, 0, 0)),
        scratch_shapes=[pltpu.VMEM((_B, _SP, _D), jnp.float32)],
    )
    out = pl.pallas_call(
        _moe_body,
        grid_spec=grid_spec,
        out_shape=jax.ShapeDtypeStruct((_B, _S, _D), jnp.float32),
        compiler_params=pltpu.CompilerParams(
            dimension_semantics=("arbitrary", "arbitrary")),
    )(pe_pad, ps_pad, pv_pad, x_p, W1, b1, W2, b2,
      gamma.reshape(1, _D), beta.reshape(1, _D))
    return out
